# baseline (speedup 1.0000x reference)
import functools

import jax
import jax.numpy as jnp
from jax import lax
from jax.experimental import pallas as pl
from jax.experimental.pallas import tpu as pltpu

B_, N_, E_, H_, W_ = 2, 16384, 256, 256, 256
NCHUNK = 2048
NBLK = N_ // NCHUNK


def _enc_body(x_ref, w1, b1, w2, b2, w3, b3, h3_ref, g_ref, gmax):
    j = pl.program_id(1)
    x = x_ref[0]
    h = jnp.maximum(jnp.dot(x, w1[...], preferred_element_type=jnp.float32) + b1[...], 0.0)
    h = jnp.maximum(jnp.dot(h, w2[...], preferred_element_type=jnp.float32) + b2[...], 0.0)
    h = jnp.dot(h, w3[...], preferred_element_type=jnp.float32) + b3[...]
    h3_ref[0] = h
    m = jnp.max(h, axis=0)

    @pl.when(j == 0)
    def _():
        gmax[...] = m

    @pl.when(j > 0)
    def _():
        gmax[...] = jnp.maximum(gmax[...], m)

    @pl.when(j == NBLK - 1)
    def _():
        g_ref[0] = jnp.broadcast_to(gmax[...][None, :], (8, E_))


def _encode(feat_ori, W1, b1, W2, b2, W3, b3):
    full = lambda shape: pl.BlockSpec(shape, lambda b, j: (0,) * len(shape))
    return pl.pallas_call(
        _enc_body,
        grid=(B_, NBLK),
        in_specs=[
            pl.BlockSpec((1, NCHUNK, 4), lambda b, j: (b, j, 0)),
            full((4, 64)), full((64,)),
            full((64, 128)), full((128,)),
            full((128, E_)), full((E_,)),
        ],
        out_specs=[
            pl.BlockSpec((1, NCHUNK, E_), lambda b, j: (b, j, 0)),
            pl.BlockSpec((1, 8, E_), lambda b, j: (b, 0, 0)),
        ],
        out_shape=[
            jax.ShapeDtypeStruct((B_, N_, E_), jnp.float32),
            jax.ShapeDtypeStruct((B_, 8, E_), jnp.float32),
        ],
        scratch_shapes=[pltpu.VMEM((E_,), jnp.float32)],
    )(feat_ori, W1, b1, W2, b2, W3, b3)


def _dec_body(h3_ref, g_ref, f_ref, w4a, w4b, b4, w5, b5,
              vals_ref, idx_ref, wts_ref):
    g = g_ref[0, 0:1]
    gb = jnp.dot(g, w4b[...], preferred_element_type=jnp.float32) + b4[...]
    h3 = h3_ref[0]
    h4 = jnp.maximum(jnp.dot(h3, w4a[...], preferred_element_type=jnp.float32) + gb, 0.0)
    vals_ref[0] = jnp.dot(h4, w5[...], preferred_element_type=jnp.float32) + b5[...]

    f = f_ref[0]
    px = f[:, 0] * (W_ - 1.0)
    py = f[:, 1] * (H_ - 1.0)
    x0 = jnp.floor(px)
    y0 = jnp.floor(py)
    wx1 = px - x0
    wx0 = 1.0 - wx1
    wy1 = py - y0
    wy0 = 1.0 - wy1
    x0i = jnp.clip(x0.astype(jnp.int32), 0, W_ - 1)
    x1i = jnp.clip((x0 + 1.0).astype(jnp.int32), 0, W_ - 1)
    y0i = jnp.clip(y0.astype(jnp.int32), 0, H_ - 1)
    y1i = jnp.clip((y0 + 1.0).astype(jnp.int32), 0, H_ - 1)
    idx_ref[0] = jnp.stack(
        [y0i * W_ + x0i, y0i * W_ + x1i, y1i * W_ + x0i, y1i * W_ + x1i], axis=0)
    wts_ref[0] = jnp.stack([wy0 * wx0, wy0 * wx1, wy1 * wx0, wy1 * wx1], axis=0)


def _decode(h3, g, feat_ori, W4a, W4b, b4, W5, b5):
    full = lambda shape: pl.BlockSpec(shape, lambda b, j: (0,) * len(shape))
    return pl.pallas_call(
        _dec_body,
        grid=(B_, NBLK),
        in_specs=[
            pl.BlockSpec((1, NCHUNK, E_), lambda b, j: (b, j, 0)),
            pl.BlockSpec((1, 8, E_), lambda b, j: (b, 0, 0)),
            pl.BlockSpec((1, NCHUNK, 4), lambda b, j: (b, j, 0)),
            full((E_, E_)), full((E_, E_)), full((E_,)),
            full((E_, E_)), full((E_,)),
        ],
        out_specs=[
            pl.BlockSpec((1, NCHUNK, E_), lambda b, j: (b, j, 0)),
            pl.BlockSpec((1, 4, NCHUNK), lambda b, j: (b, 0, j)),
            pl.BlockSpec((1, 4, NCHUNK), lambda b, j: (b, 0, j)),
        ],
        out_shape=[
            jax.ShapeDtypeStruct((B_, N_, E_), jnp.float32),
            jax.ShapeDtypeStruct((B_, 4, N_), jnp.int32),
            jax.ShapeDtypeStruct((B_, 4, N_), jnp.float32),
        ],
    )(h3, g, feat_ori, W4a, W4b, b4, W5, b5)


def kernel(feat_ori, W1, b1, W2, b2, W3, b3, W4, b4, W5, b5):
    h3, g = _encode(feat_ori, W1, b1, W2, b2, W3, b3)
    vals, idx, wts = _decode(h3, g, feat_ori, W4[:E_], W4[E_:], b4, W5, b5)
    boff = (jnp.arange(B_, dtype=jnp.int32) * (H_ * W_))[:, None, None]
    flat_idx = (idx + boff).transpose(0, 2, 1).reshape(-1)
    v = (vals[:, :, None, :] * wts.transpose(0, 2, 1)[..., None]).reshape(-1, E_)
    grid = jnp.zeros((B_ * H_ * W_, E_), jnp.float32).at[flat_idx].add(v)
    return grid.reshape(B_, H_, W_, E_).transpose(0, 3, 1, 2)


# speedup vs baseline: 1.2884x; 1.2884x over previous
import functools

import jax
import jax.numpy as jnp
from jax import lax
from jax.experimental import pallas as pl
from jax.experimental.pallas import tpu as pltpu
from jax.experimental.pallas import tpu_sc as plsc

B_, N_, E_, H_, W_ = 2, 16384, 256, 256, 256
HW_ = H_ * W_
NCHUNK = 2048
NBLK = N_ // NCHUNK
SC_NC, SC_NS = 2, 16
SC_NW = SC_NC * SC_NS
SC_CHUNK = 2048
SC_PAIRS = (B_ * E_) // SC_NW


def _enc_body(x_ref, w1, b1, w2, b2, w3, b3, h3_ref, g_ref, gmax):
    j = pl.program_id(1)
    x = x_ref[0]
    h = jnp.maximum(jnp.dot(x, w1[...], preferred_element_type=jnp.float32) + b1[...], 0.0)
    h = jnp.maximum(jnp.dot(h, w2[...], preferred_element_type=jnp.float32) + b2[...], 0.0)
    h = jnp.dot(h, w3[...], preferred_element_type=jnp.float32) + b3[...]
    h3_ref[0] = h
    m = jnp.max(h, axis=0)

    @pl.when(j == 0)
    def _():
        gmax[...] = m

    @pl.when(j > 0)
    def _():
        gmax[...] = jnp.maximum(gmax[...], m)

    @pl.when(j == NBLK - 1)
    def _():
        g_ref[0] = jnp.broadcast_to(gmax[...][None, :], (8, E_))


def _encode(feat_ori, W1, b1, W2, b2, W3, b3):
    full = lambda shape: pl.BlockSpec(shape, lambda b, j: (0,) * len(shape))
    return pl.pallas_call(
        _enc_body,
        grid=(B_, NBLK),
        in_specs=[
            pl.BlockSpec((1, NCHUNK, 4), lambda b, j: (b, j, 0)),
            full((4, 64)), full((64,)),
            full((64, 128)), full((128,)),
            full((128, E_)), full((E_,)),
        ],
        out_specs=[
            pl.BlockSpec((1, NCHUNK, E_), lambda b, j: (b, j, 0)),
            pl.BlockSpec((1, 8, E_), lambda b, j: (b, 0, 0)),
        ],
        out_shape=[
            jax.ShapeDtypeStruct((B_, N_, E_), jnp.float32),
            jax.ShapeDtypeStruct((B_, 8, E_), jnp.float32),
        ],
        scratch_shapes=[pltpu.VMEM((E_,), jnp.float32)],
    )(feat_ori, W1, b1, W2, b2, W3, b3)


def _dec_body(h3_ref, g_ref, f_ref, w4a, w4b, b4, w5, b5,
              vals_ref, idx_ref, wts_ref):
    g = g_ref[0, 0:1]
    gb = jnp.dot(g, w4b[...], preferred_element_type=jnp.float32) + b4[...]
    h3 = h3_ref[0]
    h4 = jnp.maximum(jnp.dot(h3, w4a[...], preferred_element_type=jnp.float32) + gb, 0.0)
    h5 = jnp.dot(h4, w5[...], preferred_element_type=jnp.float32) + b5[...]
    vals_ref[0] = h5.T

    f = f_ref[0]
    px = f[:, 0] * (W_ - 1.0)
    py = f[:, 1] * (H_ - 1.0)
    x0 = jnp.floor(px)
    y0 = jnp.floor(py)
    wx1 = px - x0
    wx0 = 1.0 - wx1
    wy1 = py - y0
    wy0 = 1.0 - wy1
    x0i = jnp.clip(x0.astype(jnp.int32), 0, W_ - 1)
    x1i = jnp.clip((x0 + 1.0).astype(jnp.int32), 0, W_ - 1)
    y0i = jnp.clip(y0.astype(jnp.int32), 0, H_ - 1)
    y1i = jnp.clip((y0 + 1.0).astype(jnp.int32), 0, H_ - 1)
    idx_ref[0] = jnp.stack(
        [y0i * W_ + x0i, y0i * W_ + x1i, y1i * W_ + x0i, y1i * W_ + x1i], axis=0)
    wts_ref[0] = jnp.stack([wy0 * wx0, wy0 * wx1, wy1 * wx0, wy1 * wx1], axis=0)


def _decode(h3, g, feat_ori, W4a, W4b, b4, W5, b5):
    full = lambda shape: pl.BlockSpec(shape, lambda b, j: (0,) * len(shape))
    return pl.pallas_call(
        _dec_body,
        grid=(B_, NBLK),
        in_specs=[
            pl.BlockSpec((1, NCHUNK, E_), lambda b, j: (b, j, 0)),
            pl.BlockSpec((1, 8, E_), lambda b, j: (b, 0, 0)),
            pl.BlockSpec((1, NCHUNK, 4), lambda b, j: (b, j, 0)),
            full((E_, E_)), full((E_, E_)), full((E_,)),
            full((E_, E_)), full((E_,)),
        ],
        out_specs=[
            pl.BlockSpec((1, E_, NCHUNK), lambda b, j: (b, 0, j)),
            pl.BlockSpec((1, 4, NCHUNK), lambda b, j: (b, 0, j)),
            pl.BlockSpec((1, 4, NCHUNK), lambda b, j: (b, 0, j)),
        ],
        out_shape=[
            jax.ShapeDtypeStruct((B_, E_, N_), jnp.float32),
            jax.ShapeDtypeStruct((B_, 4, N_), jnp.int32),
            jax.ShapeDtypeStruct((B_, 4, N_), jnp.float32),
        ],
    )(h3, g, feat_ori, W4a, W4b, b4, W5, b5)


def _sc_scatter_body(vals_hbm, idx_hbm, wts_hbm, out_hbm, slab, valb, idxb, wtsb):
    wid = lax.axis_index("s") * SC_NC + lax.axis_index("c")

    def per_pair(r, carry):
        p = wid * SC_PAIRS + r
        b = p // E_
        ch = p - b * E_

        def zero_body(j, c):
            slab[pl.ds(j * 16, 16)] = jnp.zeros((16,), jnp.float32)
            return c

        lax.fori_loop(0, HW_ // 16, zero_body, 0, unroll=8)

        def chunk_body(ci, c):
            e0 = ci * SC_CHUNK
            pltpu.sync_copy(vals_hbm.at[b, ch, pl.ds(e0, SC_CHUNK)], valb)
            pltpu.sync_copy(idx_hbm.at[b, :, pl.ds(e0, SC_CHUNK)], idxb)
            pltpu.sync_copy(wts_hbm.at[b, :, pl.ds(e0, SC_CHUNK)], wtsb)

            def grp_body(gi, c2):
                s = gi * 16
                v = valb[pl.ds(s, 16)]
                for kk in range(4):
                    ii = idxb[kk, pl.ds(s, 16)]
                    ww = wtsb[kk, pl.ds(s, 16)]
                    plsc.addupdate_scatter(slab, [ii], v * ww)
                return c2

            return lax.fori_loop(0, SC_CHUNK // 16, grp_body, c)

        lax.fori_loop(0, N_ // SC_CHUNK, chunk_body, 0)
        pltpu.sync_copy(slab, out_hbm.at[b, ch])
        return carry

    lax.fori_loop(0, SC_PAIRS, per_pair, 0)


def _sc_scatter(vals_t, idx, wts):
    mesh = plsc.VectorSubcoreMesh(
        core_axis_name="c", subcore_axis_name="s",
        num_cores=SC_NC, num_subcores=SC_NS)
    return pl.kernel(
        _sc_scatter_body,
        out_type=jax.ShapeDtypeStruct((B_, E_, HW_), jnp.float32),
        mesh=mesh,
        compiler_params=pltpu.CompilerParams(needs_layout_passes=False),
        scratch_types=[
            pltpu.VMEM((HW_,), jnp.float32),
            pltpu.VMEM((SC_CHUNK,), jnp.float32),
            pltpu.VMEM((4, SC_CHUNK), jnp.int32),
            pltpu.VMEM((4, SC_CHUNK), jnp.float32),
        ],
    )(vals_t, idx, wts)


def kernel(feat_ori, W1, b1, W2, b2, W3, b3, W4, b4, W5, b5):
    h3, g = _encode(feat_ori, W1, b1, W2, b2, W3, b3)
    vals_t, idx, wts = _decode(h3, g, feat_ori, W4[:E_], W4[E_:], b4, W5, b5)
    grid = _sc_scatter(vals_t, idx, wts)
    return grid.reshape(B_, E_, H_, W_)


# speedup vs baseline: 1.9314x; 1.4991x over previous
import functools

import jax
import jax.numpy as jnp
from jax import lax
from jax.experimental import pallas as pl
from jax.experimental.pallas import tpu as pltpu
from jax.experimental.pallas import tpu_sc as plsc

B_, N_, E_, H_, W_ = 2, 16384, 256, 256, 256
HW_ = H_ * W_
NCHUNK = 2048
NBLK = N_ // NCHUNK
SC_NC, SC_NS = 2, 16
SC_NW = SC_NC * SC_NS
SC_CHUNK = 2048
SC_PAIRS = (B_ * E_) // SC_NW


def _enc_body(x_ref, w1, b1, w2, b2, w3, b3, h3_ref, g_ref, gmax):
    j = pl.program_id(1)
    x = x_ref[0]
    h = jnp.maximum(jnp.dot(x, w1[...], preferred_element_type=jnp.float32) + b1[...], 0.0)
    h = jnp.maximum(jnp.dot(h, w2[...], preferred_element_type=jnp.float32) + b2[...], 0.0)
    h = jnp.dot(h, w3[...], preferred_element_type=jnp.float32) + b3[...]
    h3_ref[0] = h
    m = jnp.max(h, axis=0)

    @pl.when(j == 0)
    def _():
        gmax[...] = m

    @pl.when(j > 0)
    def _():
        gmax[...] = jnp.maximum(gmax[...], m)

    @pl.when(j == NBLK - 1)
    def _():
        g_ref[0] = jnp.broadcast_to(gmax[...][None, :], (8, E_))


def _encode(feat_ori, W1, b1, W2, b2, W3, b3):
    full = lambda shape: pl.BlockSpec(shape, lambda b, j: (0,) * len(shape))
    return pl.pallas_call(
        _enc_body,
        grid=(B_, NBLK),
        in_specs=[
            pl.BlockSpec((1, NCHUNK, 4), lambda b, j: (b, j, 0)),
            full((4, 64)), full((64,)),
            full((64, 128)), full((128,)),
            full((128, E_)), full((E_,)),
        ],
        out_specs=[
            pl.BlockSpec((1, NCHUNK, E_), lambda b, j: (b, j, 0)),
            pl.BlockSpec((1, 8, E_), lambda b, j: (b, 0, 0)),
        ],
        out_shape=[
            jax.ShapeDtypeStruct((B_, N_, E_), jnp.float32),
            jax.ShapeDtypeStruct((B_, 8, E_), jnp.float32),
        ],
        scratch_shapes=[pltpu.VMEM((E_,), jnp.float32)],
    )(feat_ori, W1, b1, W2, b2, W3, b3)


def _dec_body(h3_ref, g_ref, f_ref, w4a, w4b, b4, w5, b5,
              vals_ref, idx_ref, wts_ref):
    g = g_ref[0, 0:1]
    gb = jnp.dot(g, w4b[...], preferred_element_type=jnp.float32) + b4[...]
    h3 = h3_ref[0]
    h4 = jnp.maximum(jnp.dot(h3, w4a[...], preferred_element_type=jnp.float32) + gb, 0.0)
    h5 = jnp.dot(h4, w5[...], preferred_element_type=jnp.float32) + b5[...]
    vals_ref[0] = h5.T

    f = f_ref[0]
    px = f[:, 0] * (W_ - 1.0)
    py = f[:, 1] * (H_ - 1.0)
    x0 = jnp.floor(px)
    y0 = jnp.floor(py)
    wx1 = px - x0
    wx0 = 1.0 - wx1
    wy1 = py - y0
    wy0 = 1.0 - wy1
    x0i = jnp.clip(x0.astype(jnp.int32), 0, W_ - 1)
    x1i = jnp.clip((x0 + 1.0).astype(jnp.int32), 0, W_ - 1)
    y0i = jnp.clip(y0.astype(jnp.int32), 0, H_ - 1)
    y1i = jnp.clip((y0 + 1.0).astype(jnp.int32), 0, H_ - 1)
    idx_ref[0] = jnp.stack(
        [y0i * W_ + x0i, y0i * W_ + x1i, y1i * W_ + x0i, y1i * W_ + x1i], axis=0)
    wts_ref[0] = jnp.stack([wy0 * wx0, wy0 * wx1, wy1 * wx0, wy1 * wx1], axis=0)


def _decode(h3, g, feat_ori, W4a, W4b, b4, W5, b5):
    full = lambda shape: pl.BlockSpec(shape, lambda b, j: (0,) * len(shape))
    return pl.pallas_call(
        _dec_body,
        grid=(B_, NBLK),
        in_specs=[
            pl.BlockSpec((1, NCHUNK, E_), lambda b, j: (b, j, 0)),
            pl.BlockSpec((1, 8, E_), lambda b, j: (b, 0, 0)),
            pl.BlockSpec((1, NCHUNK, 4), lambda b, j: (b, j, 0)),
            full((E_, E_)), full((E_, E_)), full((E_,)),
            full((E_, E_)), full((E_,)),
        ],
        out_specs=[
            pl.BlockSpec((1, E_, NCHUNK), lambda b, j: (b, 0, j)),
            pl.BlockSpec((1, 4, NCHUNK), lambda b, j: (b, 0, j)),
            pl.BlockSpec((1, 4, NCHUNK), lambda b, j: (b, 0, j)),
        ],
        out_shape=[
            jax.ShapeDtypeStruct((B_, E_, N_), jnp.float32),
            jax.ShapeDtypeStruct((B_, 4, N_), jnp.int32),
            jax.ShapeDtypeStruct((B_, 4, N_), jnp.float32),
        ],
    )(h3, g, feat_ori, W4a, W4b, b4, W5, b5)


SC_NCHUNKS = N_ // SC_CHUNK


def _sc_scatter_body(vals_hbm, idx_hbm, wts_hbm, out_hbm, slab,
                     valb0, idxb0, wtsb0, sem0, valb1, idxb1, wtsb1, sem1):
    wid = lax.axis_index("s") * SC_NC + lax.axis_index("c")
    bufs = [(valb0, idxb0, wtsb0, sem0), (valb1, idxb1, wtsb1, sem1)]

    def start(b, ch, ci, slot):
        valb, idxb, wtsb, sem = bufs[slot]
        e0 = ci * SC_CHUNK
        hs = (
            pltpu.make_async_copy(vals_hbm.at[b, ch, pl.ds(e0, SC_CHUNK)], valb, sem),
            pltpu.make_async_copy(idx_hbm.at[b, :, pl.ds(e0, SC_CHUNK)], idxb, sem),
            pltpu.make_async_copy(wts_hbm.at[b, :, pl.ds(e0, SC_CHUNK)], wtsb, sem),
        )
        for h in hs:
            h.start()
        return hs

    def compute(slot):
        valb, idxb, wtsb, _ = bufs[slot]

        def grp_body(gi, c2):
            s = gi * 16
            v = valb[pl.ds(s, 16)]
            for kk in range(4):
                ii = idxb[kk, pl.ds(s, 16)]
                ww = wtsb[kk, pl.ds(s, 16)]
                plsc.addupdate_scatter(slab, [ii], v * ww)
            return c2

        lax.fori_loop(0, SC_CHUNK // 16, grp_body, 0, unroll=2)

    def per_pair(r, carry):
        p = wid * SC_PAIRS + r
        b = p // E_
        ch = p - b * E_

        hs = start(b, ch, 0, 0)

        def zero_body(j, c):
            slab[pl.ds(j * 16, 16)] = jnp.zeros((16,), jnp.float32)
            return c

        lax.fori_loop(0, HW_ // 16, zero_body, 0, unroll=8)

        for ci in range(SC_NCHUNKS):
            nxt = start(b, ch, ci + 1, (ci + 1) % 2) if ci + 1 < SC_NCHUNKS else None
            for h in hs:
                h.wait()
            compute(ci % 2)
            hs = nxt

        pltpu.sync_copy(slab, out_hbm.at[b, ch])
        return carry

    lax.fori_loop(0, SC_PAIRS, per_pair, 0)


def _sc_scatter(vals_t, idx, wts):
    mesh = plsc.VectorSubcoreMesh(
        core_axis_name="c", subcore_axis_name="s",
        num_cores=SC_NC, num_subcores=SC_NS)
    dbuf = [
        pltpu.VMEM((SC_CHUNK,), jnp.float32),
        pltpu.VMEM((4, SC_CHUNK), jnp.int32),
        pltpu.VMEM((4, SC_CHUNK), jnp.float32),
        pltpu.SemaphoreType.DMA,
    ]
    return pl.kernel(
        _sc_scatter_body,
        out_type=jax.ShapeDtypeStruct((B_, E_, HW_), jnp.float32),
        mesh=mesh,
        compiler_params=pltpu.CompilerParams(needs_layout_passes=False),
        scratch_types=[pltpu.VMEM((HW_,), jnp.float32)] + dbuf + dbuf,
    )(vals_t, idx, wts)


def kernel(feat_ori, W1, b1, W2, b2, W3, b3, W4, b4, W5, b5):
    h3, g = _encode(feat_ori, W1, b1, W2, b2, W3, b3)
    vals_t, idx, wts = _decode(h3, g, feat_ori, W4[:E_], W4[E_:], b4, W5, b5)
    grid = _sc_scatter(vals_t, idx, wts)
    return grid.reshape(B_, E_, H_, W_)
